# baseline (speedup 1.0000x reference)
import functools

import jax
import jax.numpy as jnp
from jax import lax
from jax.experimental import pallas as pl
from jax.experimental.pallas import tpu as pltpu
from jax.experimental.pallas import tpu_sc as plsc

B = 16384
L = 50
D = 16
V_HIST = 1000000
NC = 2
NS = 16
NW = NC * NS
BPW = B // NW
CB = 32
NCHUNK = BPW // CB
HIDX = CB * L

CW = 1000
NCHT = V_HIST // CW

_SC_PARAMS = pltpu.CompilerParams(use_tc_tiling_on_sc=False,
                                  needs_layout_passes=False)
_MESH = plsc.VectorSubcoreMesh(core_axis_name="c", subcore_axis_name="s",
                               num_cores=NC, num_subcores=NS)


NSLOT = 1024 // NW


def _xpose_body(tT_hbm, t_rm_out, pin0, pin1, pout0, pout1,
                sem_i0, sem_i1, sem_o):
    wid = lax.axis_index("s") * NC + lax.axis_index("c")
    lane = lax.iota(jnp.int32, 16)
    pin = (pin0, pin1)
    pout = (pout0, pout1)
    sem_i = (sem_i0, sem_i1)

    def ceff(s):
        c = wid + s * NW
        return jnp.where(c < NCHT, c, 0)

    def fire(s, par):
        c = ceff(s)
        for d in range(D):
            pltpu.async_copy(tT_hbm.at[d, pl.ds(c * CW, CW)],
                             pin[par].at[pl.ds(d * CW, CW)], sem_i[par])

    def process(s, par, drain_out):
        c = ceff(s)
        pltpu.make_async_copy(tT_hbm.at[0, pl.ds(0, D * CW)], pin[par],
                              sem_i[par]).wait()
        if drain_out:
            pltpu.make_async_copy(t_rm_out.at[pl.ds(0, CW)], pout[par],
                                  sem_o).wait()

        def v_body(g, _):
            rowi = lane + g * 16
            for d in range(D):
                x = pin[par][pl.ds(d * CW + g * 16, 16)]
                plsc.store_scatter(
                    pout[par], [rowi, jnp.full((16,), d, jnp.int32)], x)
            return 0

        lax.fori_loop(0, CW // 16, v_body, 0)
        pltpu.async_copy(pout[par], t_rm_out.at[pl.ds(c * CW, CW)], sem_o)

    fire(0, 0)
    fire(1, 1)
    process(0, 0, False)
    fire(2, 0)
    process(1, 1, False)
    fire(3, 1)

    def pair_body(i, _):
        s = 2 * i + 2
        process(s, 0, True)
        fire(s + 2, 0)
        process(s + 1, 1, True)
        fire(s + 3, 1)
        return 0

    lax.fori_loop(0, (NSLOT - 4) // 2, pair_body, 0)
    process(NSLOT - 2, 0, True)
    process(NSLOT - 1, 1, True)
    pltpu.make_async_copy(t_rm_out.at[pl.ds(0, CW)], pout0, sem_o).wait()
    pltpu.make_async_copy(t_rm_out.at[pl.ds(0, CW)], pout1, sem_o).wait()


def _embed_body(uid_hbm, hist_hbm, iid_hbm, icate_hbm,
                tuT_hbm, th_rm_hbm, tiT_hbm, tcT_hbm,
                eu_out, ei_out,
                idx_u, idx_i, idx_c, hidx0, hidx1,
                up, ip, cp, hrows0, hrows1,
                eu_buf, ei_buf,
                sem_u, sem_ic, sem_h0, sem_h1, sem_o):
    wid = lax.axis_index("s") * NC + lax.axis_index("c")
    base = wid * BPW

    pltpu.sync_copy(uid_hbm.at[pl.ds(base, BPW)], idx_u)
    pltpu.sync_copy(iid_hbm.at[pl.ds(base, BPW)], idx_i)
    pltpu.sync_copy(icate_hbm.at[pl.ds(base, BPW)], idx_c)

    NJ = BPW // 128

    @pl.loop(0, D * NJ)
    def u_issue(k):
        d = k // NJ
        j = k % NJ
        pltpu.async_copy(tuT_hbm.at[d].at[idx_u.at[pl.ds(j * 128, 128)]],
                         up.at[d, pl.ds(j * 128, 128)], sem_u)

    @pl.loop(0, D * NJ)
    def i_issue(k):
        d = k // NJ
        j = k % NJ
        pltpu.async_copy(tiT_hbm.at[d].at[idx_i.at[pl.ds(j * 128, 128)]],
                         ip.at[d, pl.ds(j * 128, 128)], sem_ic)

    @pl.loop(0, D * NJ)
    def c_issue(k):
        d = k // NJ
        j = k % NJ
        pltpu.async_copy(tcT_hbm.at[d].at[idx_c.at[pl.ds(j * 128, 128)]],
                         cp.at[d, pl.ds(j * 128, 128)], sem_ic)

    hidx = (hidx0, hidx1)
    hrows = (hrows0, hrows1)
    sem_h = (sem_h0, sem_h1)

    def fire(c, par):
        pltpu.sync_copy(hist_hbm.at[pl.ds(base + c * CB, CB)], hidx[par])

        @pl.loop(0, CB)
        def issue(b):
            pltpu.async_copy(th_rm_hbm.at[hidx[par].at[b]],
                             hrows[par].at[pl.ds(b * L, L)], sem_h[par])

    def drain(par):
        pltpu.make_async_copy(th_rm_hbm.at[pl.ds(0, HIDX)], hrows[par],
                              sem_h[par]).wait()

    zeros = jnp.zeros((D,), jnp.float32)
    ones = jnp.ones((D,), jnp.float32)
    rowidx = lax.iota(jnp.int32, 16)

    def compute(c, par):
        rows = hrows[par]

        def b_body(b, _):
            acc = zeros
            cnt = zeros
            for j in range(L):
                r = rows[b * L + j]
                acc = acc + r
                cnt = cnt + jnp.where(r != 0.0, ones, zeros)
            pooled = acc / (cnt + 1e-16)
            bb = c * CB + b
            bcol = jnp.full((16,), bb, jnp.int32)
            u = plsc.load_gather(up, [rowidx, bcol])
            iv = plsc.load_gather(ip, [rowidx, bcol])
            cv = plsc.load_gather(cp, [rowidx, bcol])
            eu_buf[bb] = u + pooled
            ei_buf[bb] = iv + cv
            return 0

        lax.fori_loop(0, CB, b_body, 0)

    fire(0, 0)
    fire(1, 1)
    pltpu.make_async_copy(tuT_hbm.at[pl.ds(0, D), pl.ds(0, BPW)],
                          up, sem_u).wait()
    pltpu.make_async_copy(tuT_hbm.at[pl.ds(0, D), pl.ds(0, BPW)],
                          ip, sem_ic).wait()
    pltpu.make_async_copy(tuT_hbm.at[pl.ds(0, D), pl.ds(0, BPW)],
                          cp, sem_ic).wait()

    def pair_body(i, _):
        c = 2 * i
        drain(0)
        compute(c, 0)
        fire(c + 2, 0)
        drain(1)
        compute(c + 1, 1)
        fire(c + 3, 1)
        return 0

    lax.fori_loop(0, NCHUNK // 2 - 1, pair_body, 0)
    drain(0)
    compute(NCHUNK - 2, 0)
    drain(1)
    compute(NCHUNK - 1, 1)

    pltpu.async_copy(eu_buf, eu_out.at[pl.ds(base, BPW)], sem_o)
    pltpu.async_copy(ei_buf, ei_out.at[pl.ds(base, BPW)], sem_o)
    pltpu.make_async_copy(th_rm_hbm.at[pl.ds(0, BPW)], eu_buf, sem_o).wait()
    pltpu.make_async_copy(th_rm_hbm.at[pl.ds(0, BPW)], ei_buf, sem_o).wait()


_sc_xpose = pl.kernel(
    _xpose_body,
    out_type=[jax.ShapeDtypeStruct((V_HIST, D), jnp.float32)],
    mesh=_MESH,
    scratch_types=[
        pltpu.VMEM((D * CW,), jnp.float32),
        pltpu.VMEM((D * CW,), jnp.float32),
        pltpu.VMEM((CW, D), jnp.float32),
        pltpu.VMEM((CW, D), jnp.float32),
        pltpu.SemaphoreType.DMA,
        pltpu.SemaphoreType.DMA,
        pltpu.SemaphoreType.DMA,
    ],
    compiler_params=_SC_PARAMS,
)

_sc_embed = pl.kernel(
    _embed_body,
    out_type=[jax.ShapeDtypeStruct((B, D), jnp.float32),
              jax.ShapeDtypeStruct((B, D), jnp.float32)],
    mesh=_MESH,
    scratch_types=[
        pltpu.VMEM((BPW,), jnp.int32),
        pltpu.VMEM((BPW,), jnp.int32),
        pltpu.VMEM((BPW,), jnp.int32),
        pltpu.VMEM((CB, L), jnp.int32),
        pltpu.VMEM((CB, L), jnp.int32),
        pltpu.VMEM((D, BPW), jnp.float32),
        pltpu.VMEM((D, BPW), jnp.float32),
        pltpu.VMEM((D, BPW), jnp.float32),
        pltpu.VMEM((HIDX, D), jnp.float32),
        pltpu.VMEM((HIDX, D), jnp.float32),
        pltpu.VMEM((BPW, D), jnp.float32),
        pltpu.VMEM((BPW, D), jnp.float32),
        pltpu.SemaphoreType.DMA,
        pltpu.SemaphoreType.DMA,
        pltpu.SemaphoreType.DMA,
        pltpu.SemaphoreType.DMA,
        pltpu.SemaphoreType.DMA,
    ],
    compiler_params=_SC_PARAMS,
)


BT = 2048


def _mlp_body(eu_ref, ei_ref,
              cw0a, cw0b, cb0, cw1, cb1, cw2, cb2,
              tw0a, tw0b, tb0, tw1, tb1, tw2, tb2,
              out_ref):
    eu = eu_ref[...]
    ei = ei_ref[...]

    def tower(w0a, w0b, b0, w1, b1, w2, b2):
        h = (jnp.dot(eu, w0a[...], preferred_element_type=jnp.float32)
             + jnp.dot(ei, w0b[...], preferred_element_type=jnp.float32)
             + b0[...])
        h = jnp.maximum(h, 0.0)
        h = jnp.dot(h, w1[...], preferred_element_type=jnp.float32) + b1[...]
        h = jnp.maximum(h, 0.0)
        return jnp.dot(h, w2[...], preferred_element_type=jnp.float32) + b2[...]

    cvr = jax.nn.sigmoid(tower(cw0a, cw0b, cb0, cw1, cb1, cw2, cb2))
    ctr = jax.nn.sigmoid(tower(tw0a, tw0b, tb0, tw1, tb1, tw2, tb2))
    out_ref[...] = jnp.concatenate([cvr, ctr, cvr * ctr], axis=1)


def _full(shape):
    nd = len(shape)
    return pl.BlockSpec(shape, lambda i: (0,) * nd)


def _mlp_call(eu, ei, cw0a, cw0b, cb0, cw1, cb1, cw2, cb2,
              tw0a, tw0b, tb0, tw1, tb1, tw2, tb2):
    wspecs = [_full(w.shape) for w in
              (cw0a, cw0b, cb0, cw1, cb1, cw2, cb2,
               tw0a, tw0b, tb0, tw1, tb1, tw2, tb2)]
    return pl.pallas_call(
        _mlp_body,
        grid=(B // BT,),
        in_specs=[pl.BlockSpec((BT, D), lambda i: (i, 0)),
                  pl.BlockSpec((BT, D), lambda i: (i, 0))] + wspecs,
        out_specs=pl.BlockSpec((BT, 3), lambda i: (i, 0)),
        out_shape=jax.ShapeDtypeStruct((B, 3), jnp.float32),
    )(eu, ei, cw0a, cw0b, cb0, cw1, cb1, cw2, cb2,
      tw0a, tw0b, tb0, tw1, tb1, tw2, tb2)


def kernel(user_id, user_hist, item_id, item_cate,
           table_user_id, table_user_hist, table_item_id, table_item_cate,
           cvr_W0, cvr_b0, cvr_W1, cvr_b1, cvr_W2, cvr_b2,
           ctr_W0, ctr_b0, ctr_W1, ctr_b1, ctr_W2, ctr_b2):
    uid = user_id.astype(jnp.int32)
    hist = user_hist.astype(jnp.int32)
    iid = item_id.astype(jnp.int32)
    icate = item_cate.astype(jnp.int32)

    tuT = table_user_id.T
    thT = table_user_hist.T
    tiT = table_item_id.T
    tcT = table_item_cate.T

    (th_rm,) = _sc_xpose(thT)
    eu, ei = _sc_embed(uid, hist, iid, icate, tuT, th_rm, tiT, tcT)

    return _mlp_call(eu, ei,
                     cvr_W0[:D], cvr_W0[D:], cvr_b0, cvr_W1, cvr_b1,
                     cvr_W2, cvr_b2,
                     ctr_W0[:D], ctr_W0[D:], ctr_b0, ctr_W1, ctr_b1,
                     ctr_W2, ctr_b2)


# speedup vs baseline: 8.0704x; 8.0704x over previous
import functools

import jax
import jax.numpy as jnp
from jax import lax
from jax.experimental import pallas as pl
from jax.experimental.pallas import tpu as pltpu
from jax.experimental.pallas import tpu_sc as plsc

B = 16384
L = 50
D = 16
V_BIG = 1000000
V_CATE = 100000
NC = 2
NS = 16
NW = NC * NS
BPW = B // NW
CB = 32
NCHUNK = BPW // CB
HIDX = CB * L

CWT = 1024
PV_BIG = 1000064
PV_CATE = 100096

_MESH = plsc.VectorSubcoreMesh(core_axis_name="c", subcore_axis_name="s",
                               num_cores=NC, num_subcores=NS)


def _xpose_body(tuT, thT, tiT, tcT,
                tu_rm, th_rm, ti_rm, tc_rm,
                pin0, pin1, pout0, pout1,
                sem_i0, sem_i1, sem_o):
    wid = lax.axis_index("s") * NC + lax.axis_index("c")
    lane = lax.iota(jnp.int32, 16)
    pin = (pin0, pin1)
    pout = (pout0, pout1)
    sem_i = (sem_i0, sem_i1)

    tables = [(tuT, tu_rm, PV_BIG), (thT, th_rm, PV_BIG),
              (tiT, ti_rm, PV_BIG), (tcT, tc_rm, PV_CATE)]

    def emit_table(src, dst, pv, first):
        nch = -(-pv // CWT)
        nslot = -(-nch // NW)
        nslot += nslot % 2

        def off_of(s):
            c = jnp.minimum(wid + s * NW, nch - 1)
            return jnp.minimum(c * CWT, pv - CWT)

        def fire(s, par):
            off = off_of(s)
            for tr in range(2):
                pltpu.async_copy(src.at[pl.ds(tr * 8, 8), pl.ds(off, CWT)],
                                 pin[par].at[pl.ds(tr * 8, 8)], sem_i[par])

        def process(s, par, drain_out):
            off = off_of(s)
            pltpu.make_async_copy(src.at[pl.ds(0, D), pl.ds(0, CWT)],
                                  pin[par], sem_i[par]).wait()
            if drain_out:
                pltpu.make_async_copy(dst.at[pl.ds(0, CWT * D)], pout[par],
                                      sem_o).wait()

            def g_body(g, _):
                dsti = (lane + g * 16) * D
                for d in range(D):
                    x = pin[par][d, pl.ds(g * 16, 16)]
                    plsc.store_scatter(pout[par], [dsti + d], x)
                return 0

            lax.fori_loop(0, CWT // 16, g_body, 0)
            pltpu.async_copy(pout[par], dst.at[pl.ds(off * D, CWT * D)],
                             sem_o)

        fire(0, 0)
        fire(1, 1)
        process(0, 0, not first)
        fire(2, 0)
        process(1, 1, not first)
        fire(3, 1)

        def pair_body(i, _):
            s = 2 * i + 2
            process(s, 0, True)
            fire(s + 2, 0)
            process(s + 1, 1, True)
            fire(s + 3, 1)
            return 0

        lax.fori_loop(0, (nslot - 4) // 2, pair_body, 0)
        process(nslot - 2, 0, True)
        process(nslot - 1, 1, True)

    for t, (src, dst, pv) in enumerate(tables):
        emit_table(src, dst, pv, t == 0)
    pltpu.make_async_copy(tu_rm.at[pl.ds(0, CWT * D)], pout0, sem_o).wait()
    pltpu.make_async_copy(tu_rm.at[pl.ds(0, CWT * D)], pout1, sem_o).wait()


def _embed_body(uid_hbm, hist_hbm, iid_hbm, icate_hbm,
                tu_hbm, th_hbm, ti_hbm, tc_hbm,
                eu_out, ei_out,
                idx_u, idx_i, idx_c, hidx0, hidx1,
                urows, irows, crows, hrows0, hrows1,
                eu_buf, ei_buf,
                sem_r, sem_h0, sem_h1, sem_o):
    wid = lax.axis_index("s") * NC + lax.axis_index("c")
    base = wid * BPW

    pltpu.sync_copy(uid_hbm.at[pl.ds(base, BPW)], idx_u)
    pltpu.sync_copy(iid_hbm.at[pl.ds(base, BPW)], idx_i)
    pltpu.sync_copy(icate_hbm.at[pl.ds(base, BPW)], idx_c)
    cps = []
    for j in range(BPW // 128):
        cps.append(pltpu.async_copy(
            tu_hbm.at[idx_u.at[pl.ds(j * 128, 128)]],
            urows.at[pl.ds(j * 128, 128)], sem_r))
        cps.append(pltpu.async_copy(
            ti_hbm.at[idx_i.at[pl.ds(j * 128, 128)]],
            irows.at[pl.ds(j * 128, 128)], sem_r))
        cps.append(pltpu.async_copy(
            tc_hbm.at[idx_c.at[pl.ds(j * 128, 128)]],
            crows.at[pl.ds(j * 128, 128)], sem_r))

    hidx = (hidx0, hidx1)
    hrows = (hrows0, hrows1)
    sem_h = (sem_h0, sem_h1)

    def fire(c, par):
        pltpu.sync_copy(hist_hbm.at[pl.ds(base + c * CB, CB)], hidx[par])

        @pl.loop(0, CB)
        def issue(b):
            pltpu.async_copy(th_hbm.at[hidx[par].at[b]],
                             hrows[par].at[pl.ds(b * L, L)], sem_h[par])

    def drain(par):
        pltpu.make_async_copy(th_hbm.at[pl.ds(0, HIDX)], hrows[par],
                              sem_h[par]).wait()

    zeros = jnp.zeros((D,), jnp.float32)
    ones = jnp.ones((D,), jnp.float32)

    def compute(c, par):
        rows = hrows[par]

        def b_body(b, _):
            acc = zeros
            cnt = zeros
            for j in range(L):
                r = rows[b * L + j]
                acc = acc + r
                cnt = cnt + jnp.where(r != 0.0, ones, zeros)
            pooled = acc / (cnt + 1e-16)
            bb = c * CB + b
            eu_buf[bb] = urows[bb] + pooled
            ei_buf[bb] = irows[bb] + crows[bb]
            return 0

        lax.fori_loop(0, CB, b_body, 0)

    fire(0, 0)
    fire(1, 1)
    for cp in cps:
        cp.wait()

    def pair_body(i, _):
        c = 2 * i
        drain(0)
        compute(c, 0)
        fire(c + 2, 0)
        drain(1)
        compute(c + 1, 1)
        fire(c + 3, 1)
        return 0

    lax.fori_loop(0, NCHUNK // 2 - 1, pair_body, 0)
    drain(0)
    compute(NCHUNK - 2, 0)
    drain(1)
    compute(NCHUNK - 1, 1)

    pltpu.async_copy(eu_buf, eu_out.at[pl.ds(base, BPW)], sem_o)
    pltpu.async_copy(ei_buf, ei_out.at[pl.ds(base, BPW)], sem_o)
    pltpu.make_async_copy(th_hbm.at[pl.ds(0, BPW)], eu_buf, sem_o).wait()
    pltpu.make_async_copy(th_hbm.at[pl.ds(0, BPW)], ei_buf, sem_o).wait()


_sc_xpose = pl.kernel(
    _xpose_body,
    out_type=[jax.ShapeDtypeStruct((PV_BIG * D,), jnp.float32),
              jax.ShapeDtypeStruct((PV_BIG * D,), jnp.float32),
              jax.ShapeDtypeStruct((PV_BIG * D,), jnp.float32),
              jax.ShapeDtypeStruct((PV_CATE * D,), jnp.float32)],
    mesh=_MESH,
    scratch_types=[
        pltpu.VMEM((D, CWT), jnp.float32),
        pltpu.VMEM((D, CWT), jnp.float32),
        pltpu.VMEM((CWT * D,), jnp.float32),
        pltpu.VMEM((CWT * D,), jnp.float32),
        pltpu.SemaphoreType.DMA,
        pltpu.SemaphoreType.DMA,
        pltpu.SemaphoreType.DMA,
    ],
    compiler_params=pltpu.CompilerParams(use_tc_tiling_on_sc=True,
                                         needs_layout_passes=False),
)

_sc_embed = pl.kernel(
    _embed_body,
    out_type=[jax.ShapeDtypeStruct((B, D), jnp.float32),
              jax.ShapeDtypeStruct((B, D), jnp.float32)],
    mesh=_MESH,
    scratch_types=[
        pltpu.VMEM((BPW,), jnp.int32),
        pltpu.VMEM((BPW,), jnp.int32),
        pltpu.VMEM((BPW,), jnp.int32),
        pltpu.VMEM((CB, L), jnp.int32),
        pltpu.VMEM((CB, L), jnp.int32),
        pltpu.VMEM((BPW, D), jnp.float32),
        pltpu.VMEM((BPW, D), jnp.float32),
        pltpu.VMEM((BPW, D), jnp.float32),
        pltpu.VMEM((HIDX, D), jnp.float32),
        pltpu.VMEM((HIDX, D), jnp.float32),
        pltpu.VMEM((BPW, D), jnp.float32),
        pltpu.VMEM((BPW, D), jnp.float32),
        pltpu.SemaphoreType.DMA,
        pltpu.SemaphoreType.DMA,
        pltpu.SemaphoreType.DMA,
        pltpu.SemaphoreType.DMA,
    ],
    compiler_params=pltpu.CompilerParams(use_tc_tiling_on_sc=False),
)


BT = 2048


def _mlp_body(eu_ref, ei_ref,
              cw0a, cw0b, cb0, cw1, cb1, cw2, cb2,
              tw0a, tw0b, tb0, tw1, tb1, tw2, tb2,
              out_ref):
    eu = eu_ref[...]
    ei = ei_ref[...]

    def tower(w0a, w0b, b0, w1, b1, w2, b2):
        h = (jnp.dot(eu, w0a[...], preferred_element_type=jnp.float32)
             + jnp.dot(ei, w0b[...], preferred_element_type=jnp.float32)
             + b0[...])
        h = jnp.maximum(h, 0.0)
        h = jnp.dot(h, w1[...], preferred_element_type=jnp.float32) + b1[...]
        h = jnp.maximum(h, 0.0)
        return jnp.dot(h, w2[...], preferred_element_type=jnp.float32) + b2[...]

    cvr = jax.nn.sigmoid(tower(cw0a, cw0b, cb0, cw1, cb1, cw2, cb2))
    ctr = jax.nn.sigmoid(tower(tw0a, tw0b, tb0, tw1, tb1, tw2, tb2))
    out_ref[...] = jnp.concatenate([cvr, ctr, cvr * ctr], axis=1)


def _full(shape):
    nd = len(shape)
    return pl.BlockSpec(shape, lambda i: (0,) * nd)


def _mlp_call(eu, ei, cw0a, cw0b, cb0, cw1, cb1, cw2, cb2,
              tw0a, tw0b, tb0, tw1, tb1, tw2, tb2):
    wspecs = [_full(w.shape) for w in
              (cw0a, cw0b, cb0, cw1, cb1, cw2, cb2,
               tw0a, tw0b, tb0, tw1, tb1, tw2, tb2)]
    return pl.pallas_call(
        _mlp_body,
        grid=(B // BT,),
        in_specs=[pl.BlockSpec((BT, D), lambda i: (i, 0)),
                  pl.BlockSpec((BT, D), lambda i: (i, 0))] + wspecs,
        out_specs=pl.BlockSpec((BT, 3), lambda i: (i, 0)),
        out_shape=jax.ShapeDtypeStruct((B, 3), jnp.float32),
    )(eu, ei, cw0a, cw0b, cb0, cw1, cb1, cw2, cb2,
      tw0a, tw0b, tb0, tw1, tb1, tw2, tb2)


def kernel(user_id, user_hist, item_id, item_cate,
           table_user_id, table_user_hist, table_item_id, table_item_cate,
           cvr_W0, cvr_b0, cvr_W1, cvr_b1, cvr_W2, cvr_b2,
           ctr_W0, ctr_b0, ctr_W1, ctr_b1, ctr_W2, ctr_b2):
    uid = user_id.astype(jnp.int32)
    hist = user_hist.astype(jnp.int32)
    iid = item_id.astype(jnp.int32)
    icate = item_cate.astype(jnp.int32)

    tu_rm, th_rm, ti_rm, tc_rm = _sc_xpose(
        table_user_id.T, table_user_hist.T,
        table_item_id.T, table_item_cate.T)

    eu, ei = _sc_embed(uid, hist, iid, icate,
                       tu_rm.reshape(PV_BIG, D), th_rm.reshape(PV_BIG, D),
                       ti_rm.reshape(PV_BIG, D), tc_rm.reshape(PV_CATE, D))

    return _mlp_call(eu, ei,
                     cvr_W0[:D], cvr_W0[D:], cvr_b0, cvr_W1, cvr_b1,
                     cvr_W2, cvr_b2,
                     ctr_W0[:D], ctr_W0[D:], ctr_b0, ctr_W1, ctr_b1,
                     ctr_W2, ctr_b2)
